# baseline (speedup 1.0000x reference)
import functools

import jax
import jax.numpy as jnp
from jax import lax
from jax.experimental import pallas as pl
from jax.experimental.pallas import tpu as pltpu
from jax.experimental.pallas import tpu_sc as plsc

_D = 128
_CH = 256


def _make_sc_kernel(n_img, positions):
    info = plsc.get_sparse_core_info()
    NC, NS = info.num_cores, info.num_subcores
    NW = NC * NS
    B = n_img * positions
    b_per_w = B // NW
    n_chunks = b_per_w // _CH
    w_per_img = positions // b_per_w
    assert b_per_w % _CH == 0 and positions % b_per_w == 0
    mesh = plsc.VectorSubcoreMesh(core_axis_name="c", subcore_axis_name="s")

    @functools.partial(
        pl.kernel, mesh=mesh,
        out_type=jax.ShapeDtypeStruct((n_img, _D, positions), jnp.float32),
        compiler_params=pltpu.CompilerParams(needs_layout_passes=False),
        scratch_types=[
            pltpu.VMEM((_CH,), jnp.int32),
            pltpu.VMEM((_CH, _D), jnp.float32),
            pltpu.VMEM((_D, _CH + 1), jnp.float32),
            pltpu.SemaphoreType.DMA,
        ],
    )
    def k(idx_hbm, table_hbm, out_hbm, idx_v, rows_v, cols_v, sem):
        wid = lax.axis_index("s") * NC + lax.axis_index("c")
        nimg = wid // w_per_img
        pbase = (wid % w_per_img) * b_per_w
        lane = lax.broadcasted_iota(jnp.int32, (16,), 0)

        def body(i, carry):
            poff = pbase + i * _CH
            pltpu.sync_copy(idx_hbm.at[pl.ds(wid * b_per_w + i * _CH, _CH)], idx_v)
            pltpu.async_copy(table_hbm.at[idx_v], rows_v, sem).wait()

            @plsc.parallel_loop(0, _CH, 1, unroll=4)
            def tr_body(p):
                pvec = jnp.full((16,), p, jnp.int32)
                for c0 in range(0, _D, 16):
                    v = rows_v[p, pl.ds(c0, 16)] + 1.0
                    plsc.store_scatter(cols_v, [c0 + lane, pvec], v)
            pltpu.sync_copy(
                cols_v.at[:, pl.ds(0, _CH)],
                out_hbm.at[nimg, :, pl.ds(poff, _CH)],
            )
            return carry

        lax.fori_loop(0, n_chunks, body, 0)

    return k


def kernel(x, pe):
    n, _, h, w = x.shape
    positions = h * w
    idx = x.reshape(n * positions)
    out = _make_sc_kernel(n, positions)(idx, pe)
    return out.reshape(n, _D, h, w)


# speedup vs baseline: 2.6051x; 2.6051x over previous
import math

import numpy as np
import jax
import jax.numpy as jnp
from jax.experimental import pallas as pl
from jax.experimental.pallas import tpu as pltpu

_NUM_EMBEDDED = 128
_TWO_PI = 2.0 * math.pi


def _split_f32(value, keep_bits):
    f = np.float32(value)
    u = np.array(f).view(np.uint32)
    u = u & np.uint32((0xFFFFFFFF << (23 - keep_bits)) & 0xFFFFFFFF)
    return float(u.view(np.float32))


_C1 = _split_f32(_TWO_PI, 10)
_C2 = _split_f32(_TWO_PI - _C1, 10)
_C3 = float(np.float32(_TWO_PI - _C1 - np.float64(_C2)))
_INV_2PI = float(np.float32(1.0 / _TWO_PI))

_NCOEF = 8


def _fit_cheb(fn, lo, hi, deg):
    j = np.arange(16 * (deg + 1))
    xs = 0.5 * (lo + hi) + 0.5 * (hi - lo) * np.cos(
        (2 * j + 1) * np.pi / (2 * len(j))
    )
    return np.polyfit(xs, fn(xs), deg)[::-1]


_UMAX = float((np.pi * 1.0005) ** 2)
_SIN_COEF = _fit_cheb(lambda u: np.sinc(np.sqrt(u) / np.pi), 0.0, _UMAX, _NCOEF - 1)
_COS_COEF = _fit_cheb(lambda u: np.cos(np.sqrt(u)), 0.0, _UMAX, _NCOEF - 1)


def _coef_plane():
    plane = np.zeros((_NUM_EMBEDDED, 128), np.float32)
    for j in range(_NCOEF):
        plane[0::2, j] = np.float32(_SIN_COEF[j])
        plane[1::2, j] = np.float32(_COS_COEF[j])
    plane[0::2, _NCOEF] = 1.0
    plane[1::2, _NCOEF + 1] = 1.0
    return plane


def _sine_enc_kernel(x_ref, freq_ref, coef_ref, o_ref):
    xf = x_ref[0, 0, 0, :].astype(jnp.float32)
    freq = freq_ref[:, 0:1]
    t = freq * xf[None, :]
    k = jnp.round(t * _INV_2PI)
    r = ((t - k * _C1) - k * _C2) - k * _C3
    u = r * r
    acc = coef_ref[:, _NCOEF - 1:_NCOEF]
    for j in range(_NCOEF - 2, -1, -1):
        acc = acc * u + coef_ref[:, j:j + 1]
    pm = coef_ref[:, _NCOEF:_NCOEF + 1]
    qm = coef_ref[:, _NCOEF + 1:_NCOEF + 2]
    w = pm * r + qm
    o_ref[0] = w * acc + 1.0


def kernel(x, pe):
    n, _, h, w = x.shape
    positions = h * w
    tile = 1792
    nblocks = positions // tile
    x4 = x.reshape(n, nblocks, 1, tile)

    div_term = jnp.exp(
        jnp.arange(0, _NUM_EMBEDDED, 2, dtype=jnp.float32)
        * (-math.log(10000.0) / _NUM_EMBEDDED)
    )
    freq = jnp.broadcast_to(
        jnp.repeat(div_term, 2)[:, None], (_NUM_EMBEDDED, 128)
    )
    coef = jnp.asarray(_coef_plane())

    out = pl.pallas_call(
        _sine_enc_kernel,
        grid=(n, nblocks),
        in_specs=[
            pl.BlockSpec((1, 1, 1, tile), lambda i, j: (i, j, 0, 0)),
            pl.BlockSpec((_NUM_EMBEDDED, 128), lambda i, j: (0, 0)),
            pl.BlockSpec((_NUM_EMBEDDED, 128), lambda i, j: (0, 0)),
        ],
        out_specs=pl.BlockSpec((1, _NUM_EMBEDDED, tile), lambda i, j: (i, 0, j)),
        out_shape=jax.ShapeDtypeStruct((n, _NUM_EMBEDDED, positions), jnp.float32),
        compiler_params=pltpu.CompilerParams(
            dimension_semantics=("parallel", "parallel")
        ),
    )(x4, freq, coef)
    return out.reshape(n, _NUM_EMBEDDED, h, w)


# speedup vs baseline: 2.6274x; 1.0086x over previous
import math

import numpy as np
import jax
import jax.numpy as jnp
from jax.experimental import pallas as pl
from jax.experimental.pallas import tpu as pltpu

_NUM_EMBEDDED = 128
_TWO_PI = 2.0 * math.pi


def _split_f32(value, keep_bits):
    f = np.float32(value)
    u = np.array(f).view(np.uint32)
    u = u & np.uint32((0xFFFFFFFF << (23 - keep_bits)) & 0xFFFFFFFF)
    return float(u.view(np.float32))


_C1 = _split_f32(_TWO_PI, 10)
_C2 = _split_f32(_TWO_PI - _C1, 10)
_C3 = float(np.float32(_TWO_PI - _C1 - np.float64(_C2)))
_INV_2PI = float(np.float32(1.0 / _TWO_PI))

_NCOEF = 8


def _fit_cheb(fn, lo, hi, deg):
    j = np.arange(16 * (deg + 1))
    xs = 0.5 * (lo + hi) + 0.5 * (hi - lo) * np.cos(
        (2 * j + 1) * np.pi / (2 * len(j))
    )
    return np.polyfit(xs, fn(xs), deg)[::-1]


_UMAX = float((np.pi * 1.0005) ** 2)
_SIN_COEF = _fit_cheb(lambda u: np.sinc(np.sqrt(u) / np.pi), 0.0, _UMAX, _NCOEF - 1)
_COS_COEF = _fit_cheb(lambda u: np.cos(np.sqrt(u)), 0.0, _UMAX, _NCOEF - 1)


def _coef_plane():
    plane = np.zeros((_NUM_EMBEDDED, 128), np.float32)
    for j in range(_NCOEF):
        plane[0::2, j] = np.float32(_SIN_COEF[j])
        plane[1::2, j] = np.float32(_COS_COEF[j])
    return plane


def _sine_enc_kernel(x_ref, freq_ref, coef_ref, o_ref):
    xf = x_ref[0, 0, 0, :].astype(jnp.float32)
    freq = freq_ref[:, 0:1]
    t = freq * xf[None, :]
    k = jnp.round(t * _INV_2PI)
    r = ((t - k * _C1) - k * _C2) - k * _C3
    u = r * r
    acc = coef_ref[:, _NCOEF - 1:_NCOEF]
    for j in range(_NCOEF - 2, -1, -1):
        acc = acc * u + coef_ref[:, j:j + 1]
    row = jax.lax.broadcasted_iota(jnp.int32, t.shape, 0)
    w = jnp.where((row % 2) == 0, r, 1.0)
    o_ref[0] = 1.0 + w * acc


def kernel(x, pe):
    n, _, h, w = x.shape
    positions = h * w
    tile = 1792
    nblocks = positions // tile
    x4 = x.reshape(n, nblocks, 1, tile)

    div_term = jnp.exp(
        jnp.arange(0, _NUM_EMBEDDED, 2, dtype=jnp.float32)
        * (-math.log(10000.0) / _NUM_EMBEDDED)
    )
    freq = jnp.broadcast_to(
        jnp.repeat(div_term, 2)[:, None], (_NUM_EMBEDDED, 128)
    )
    coef = jnp.asarray(_coef_plane())

    out = pl.pallas_call(
        _sine_enc_kernel,
        grid=(n, nblocks),
        in_specs=[
            pl.BlockSpec((1, 1, 1, tile), lambda i, j: (i, j, 0, 0)),
            pl.BlockSpec((_NUM_EMBEDDED, 128), lambda i, j: (0, 0)),
            pl.BlockSpec((_NUM_EMBEDDED, 128), lambda i, j: (0, 0)),
        ],
        out_specs=pl.BlockSpec((1, _NUM_EMBEDDED, tile), lambda i, j: (i, 0, j)),
        out_shape=jax.ShapeDtypeStruct((n, _NUM_EMBEDDED, positions), jnp.float32),
        compiler_params=pltpu.CompilerParams(
            dimension_semantics=("parallel", "parallel")
        ),
    )(x4, freq, coef)
    return out.reshape(n, _NUM_EMBEDDED, h, w)


# speedup vs baseline: 2.6311x; 1.0014x over previous
import math

import numpy as np
import jax
import jax.numpy as jnp
from jax.experimental import pallas as pl
from jax.experimental.pallas import tpu as pltpu

_NUM_EMBEDDED = 128
_TWO_PI = 2.0 * math.pi


def _split_f32(value, keep_bits):
    f = np.float32(value)
    u = np.array(f).view(np.uint32)
    u = u & np.uint32((0xFFFFFFFF << (23 - keep_bits)) & 0xFFFFFFFF)
    return float(u.view(np.float32))


_C1 = _split_f32(_TWO_PI, 10)
_C2 = _split_f32(_TWO_PI - _C1, 10)
_C3 = float(np.float32(_TWO_PI - _C1 - np.float64(_C2)))
_INV_2PI = float(np.float32(1.0 / _TWO_PI))

_NCOEF = 8


def _fit_cheb(fn, lo, hi, deg):
    j = np.arange(16 * (deg + 1))
    xs = 0.5 * (lo + hi) + 0.5 * (hi - lo) * np.cos(
        (2 * j + 1) * np.pi / (2 * len(j))
    )
    return np.polyfit(xs, fn(xs), deg)[::-1]


_UMAX = float((np.pi * 1.0005) ** 2)
_SIN_COEF = _fit_cheb(lambda u: np.sinc(np.sqrt(u) / np.pi), 0.0, _UMAX, _NCOEF - 1)
_COS_COEF = _fit_cheb(lambda u: np.cos(np.sqrt(u)), 0.0, _UMAX, _NCOEF - 1)


def _coef_plane():
    plane = np.zeros((_NUM_EMBEDDED, 128), np.float32)
    for j in range(_NCOEF):
        plane[0::2, j] = np.float32(_SIN_COEF[j])
        plane[1::2, j] = np.float32(_COS_COEF[j])
    return plane


def _sine_enc_kernel(x_ref, freq_ref, coef_ref, o_ref):
    xf = x_ref[0, 0, 0, :].astype(jnp.float32)
    freq = freq_ref[:, 0:1]
    t = freq * xf[None, :]
    k = jnp.round(t * _INV_2PI)
    r = ((t - k * _C1) - k * _C2) - k * _C3
    u = r * r
    acc = coef_ref[:, _NCOEF - 1:_NCOEF]
    for j in range(_NCOEF - 2, -1, -1):
        acc = acc * u + coef_ref[:, j:j + 1]
    row = jax.lax.broadcasted_iota(jnp.int32, t.shape, 0)
    w = jnp.where((row % 2) == 0, r, 1.0)
    o_ref[0] = 1.0 + w * acc


def kernel(x, pe):
    n, _, h, w = x.shape
    positions = h * w
    tile = 1792
    nblocks = positions // tile
    x4 = x.reshape(n, nblocks, 1, tile)

    div_term = jnp.exp(
        jnp.arange(0, _NUM_EMBEDDED, 2, dtype=jnp.float32)
        * (-math.log(10000.0) / _NUM_EMBEDDED)
    )
    freq = jnp.broadcast_to(
        jnp.repeat(div_term, 2)[:, None], (_NUM_EMBEDDED, 128)
    )
    coef = jnp.asarray(_coef_plane())

    out = pl.pallas_call(
        _sine_enc_kernel,
        grid=(nblocks, n),
        in_specs=[
            pl.BlockSpec((1, 1, 1, tile), lambda j, i: (i, j, 0, 0)),
            pl.BlockSpec((_NUM_EMBEDDED, 128), lambda j, i: (0, 0)),
            pl.BlockSpec((_NUM_EMBEDDED, 128), lambda j, i: (0, 0)),
        ],
        out_specs=pl.BlockSpec((1, _NUM_EMBEDDED, tile), lambda j, i: (i, 0, j)),
        out_shape=jax.ShapeDtypeStruct((n, _NUM_EMBEDDED, positions), jnp.float32),
        compiler_params=pltpu.CompilerParams(
            dimension_semantics=("parallel", "parallel")
        ),
    )(x4, freq, coef)
    return out.reshape(n, _NUM_EMBEDDED, h, w)
